# Initial kernel scaffold; baseline (speedup 1.0000x reference)
import jax
import jax.numpy as jnp
from jax.experimental import pallas as pl


def kernel(x, adj, w, b):
    raise NotImplementedError("write your pallas kernel here")


# speedup vs baseline: 1.3277x; 1.3277x over previous
import jax
import jax.numpy as jnp
from jax.experimental import pallas as pl
from jax.experimental.pallas import tpu as pltpu


def _gcn_kernel(x_ref, adj_ref, w_ref, b_ref, o_ref):
    y = jnp.dot(x_ref[...].astype(jnp.bfloat16), w_ref[...],
                preferred_element_type=jnp.float32) + b_ref[...]
    z = jnp.dot(adj_ref[...], y.astype(jnp.bfloat16),
                preferred_element_type=jnp.float32)
    o_ref[...] = jnp.maximum(z, 0.0)


def kernel(x, adj, w, b):
    S, B, H = x.shape
    O = w.shape[0]

    x2 = x.reshape(S, B * H)
    adj_bm = jnp.transpose(adj.astype(jnp.bfloat16), (2, 0, 1))
    w_t = jnp.transpose(w).astype(jnp.bfloat16)
    b2d = b.reshape(1, O).astype(jnp.float32)

    out2 = pl.pallas_call(
        _gcn_kernel,
        out_shape=jax.ShapeDtypeStruct((S, B * O), jnp.float32),
        grid_spec=pltpu.PrefetchScalarGridSpec(
            num_scalar_prefetch=0,
            grid=(B,),
            in_specs=[
                pl.BlockSpec((S, H), lambda i: (0, i)),
                pl.BlockSpec((None, S, S), lambda i: (i, 0, 0)),
                pl.BlockSpec((H, O), lambda i: (0, 0)),
                pl.BlockSpec((1, O), lambda i: (0, 0)),
            ],
            out_specs=pl.BlockSpec((S, O), lambda i: (0, i)),
        ),
        compiler_params=pltpu.CompilerParams(
            dimension_semantics=("parallel",),
            vmem_limit_bytes=64 * 1024 * 1024,
        ),
    )(x2, adj_bm, w_t, b2d)

    return out2.reshape(S, B, O)
